# baseline (speedup 1.0000x reference)
import jax
import jax.numpy as jnp
from jax.experimental import pallas as pl
from jax.experimental.pallas import tpu as pltpu
from functools import partial

B, C, H, W = 8, 384, 32, 32
L = H * W
N = 4
DI = 2 * C
R = -(-C // 16)
K = 4
HID = 4 * C
LC = 128
NC = L // LC
BM = 1024
M = B * L


def _ln(x, g, b, eps=1e-5):
    mu = jnp.mean(x, -1, keepdims=True)
    var = jnp.mean((x - mu) ** 2, -1, keepdims=True)
    return (x - mu) * jax.lax.rsqrt(var + eps) * g + b


def _pre_kernel(xin_ref, g_ref, b_ref, wip_ref, cw_ref, cb_ref,
                xc_ref, z_ref, pad_ref):
    xb = xin_ref[0]
    xn = _ln(xb, g_ref[0], b_ref[0])
    xz = jnp.dot(xn, wip_ref[...], preferred_element_type=jnp.float32)
    z_ref[0] = xz[:, DI:]
    pad_ref[...] = jnp.zeros_like(pad_ref)
    pad_ref[1:H + 1, 1:W + 1, :] = xz[:, :DI].reshape(H, W, DI)
    y = cb_ref[0] * jnp.ones((H, W, DI), jnp.float32)
    for i in range(3):
        for j in range(3):
            y = y + pad_ref[i:i + H, j:j + W, :] * cw_ref[i * 3 + j]
    xc_ref[0] = (y * jax.nn.sigmoid(y)).reshape(L, DI)


def _pre(xin, ln1_g, ln1_b, wip_T, cw9, cb):
    return pl.pallas_call(
        _pre_kernel,
        grid=(B,),
        in_specs=[
            pl.BlockSpec((1, L, C), lambda i: (i, 0, 0)),
            pl.BlockSpec((1, C), lambda i: (0, 0)),
            pl.BlockSpec((1, C), lambda i: (0, 0)),
            pl.BlockSpec((C, 2 * DI), lambda i: (0, 0)),
            pl.BlockSpec((16, DI), lambda i: (0, 0)),
            pl.BlockSpec((1, DI), lambda i: (0, 0)),
        ],
        out_specs=[
            pl.BlockSpec((1, L, DI), lambda i: (i, 0, 0)),
            pl.BlockSpec((1, L, DI), lambda i: (i, 0, 0)),
        ],
        out_shape=[
            jax.ShapeDtypeStruct((B, L, DI), jnp.float32),
            jax.ShapeDtypeStruct((B, L, DI), jnp.float32),
        ],
        scratch_shapes=[pltpu.VMEM((H + 2, W + 2, DI), jnp.float32)],
        compiler_params=pltpu.CompilerParams(
            dimension_semantics=("parallel",)),
        name="mamba_pre",
    )(xin, ln1_g, ln1_b, wip_T, cw9, cb)


def _scan_kernel(x_ref, wdt_ref, wbc_ref, dtw_ref, dtb_ref, al_ref, ds_ref,
                 y_ref, h_ref, a_ref, du_ref, bc_ref):
    c = pl.program_id(1)
    xb = x_ref[0]
    x2 = xb.reshape(LC * B, DI)
    dts = jnp.dot(x2, wdt_ref[0], preferred_element_type=jnp.float32)
    dt = jax.nn.softplus(
        jnp.dot(dts, dtw_ref[0], preferred_element_type=jnp.float32)
        + dtb_ref[0])
    bc = jnp.dot(x2, wbc_ref[0], preferred_element_type=jnp.float32)
    bc_ref[...] = bc.reshape(LC, B, 2 * N)
    du_ref[...] = (dt * x2).reshape(LC, B, DI)
    A = -jnp.exp(al_ref[0])
    for n in range(N):
        a_ref[n] = jnp.exp(dt * A[n:n + 1]).reshape(LC, B, DI)

    @pl.when(c == 0)
    def _():
        h_ref[...] = jnp.zeros_like(h_ref)

    def body(t, hs):
        du = du_ref[t]
        bct = bc_ref[t]
        acc = jnp.zeros((B, DI), jnp.float32)
        new = []
        for n in range(N):
            hn = a_ref[n, t] * hs[n] + du * bct[:, n:n + 1]
            acc = acc + hn * bct[:, N + n:N + n + 1]
            new.append(hn)
        y_ref[0, t] = acc
        return tuple(new)

    hs0 = tuple(h_ref[n] for n in range(N))
    hs = jax.lax.fori_loop(0, LC, body, hs0)
    for n in range(N):
        h_ref[n] = hs[n]
    y_ref[0] += xb * ds_ref[0]


def _scan(xs4, wdt_T, wbc_T, dtw_T, dtb2, alT, ds2):
    return pl.pallas_call(
        _scan_kernel,
        grid=(K, NC),
        in_specs=[
            pl.BlockSpec((1, LC, B, DI), lambda k, c: (k, c, 0, 0)),
            pl.BlockSpec((1, DI, R), lambda k, c: (k, 0, 0)),
            pl.BlockSpec((1, DI, 2 * N), lambda k, c: (k, 0, 0)),
            pl.BlockSpec((1, R, DI), lambda k, c: (k, 0, 0)),
            pl.BlockSpec((1, 1, DI), lambda k, c: (k, 0, 0)),
            pl.BlockSpec((1, N, DI), lambda k, c: (k, 0, 0)),
            pl.BlockSpec((1, 1, DI), lambda k, c: (k, 0, 0)),
        ],
        out_specs=pl.BlockSpec((1, LC, B, DI), lambda k, c: (k, c, 0, 0)),
        out_shape=jax.ShapeDtypeStruct((K, L, B, DI), jnp.float32),
        scratch_shapes=[
            pltpu.VMEM((N, B, DI), jnp.float32),
            pltpu.VMEM((N, LC, B, DI), jnp.float32),
            pltpu.VMEM((LC, B, DI), jnp.float32),
            pltpu.VMEM((LC, B, 2 * N), jnp.float32),
        ],
        compiler_params=pltpu.CompilerParams(
            dimension_semantics=("parallel", "arbitrary")),
        name="mamba_scan",
    )(xs4, wdt_T, wbc_T, dtw_T, dtb2, alT, ds2)


def _post_kernel(y_ref, z_ref, sc_ref, ong_ref, onb_ref, wout_ref,
                 l2g_ref, l2b_ref, w1_ref, b1_ref, w2_ref, b2_ref, o_ref):
    y = _ln(y_ref[...], ong_ref[0], onb_ref[0])
    z = z_ref[...]
    y = y * (z * jax.nn.sigmoid(z))
    u = jnp.dot(y, wout_ref[...], preferred_element_type=jnp.float32)
    xr = sc_ref[...] + u
    m = _ln(xr, l2g_ref[0], l2b_ref[0])
    m = jax.nn.gelu(
        jnp.dot(m, w1_ref[...], preferred_element_type=jnp.float32)
        + b1_ref[0])
    m = jnp.dot(m, w2_ref[...], preferred_element_type=jnp.float32) + b2_ref[0]
    o_ref[...] = xr + m


def _post(y, z, sc, ong, onb, wout_T, l2g, l2b, w1_T, b1, w2_T, b2):
    row = lambda i: (i, 0)
    fix = lambda i: (0, 0)
    return pl.pallas_call(
        _post_kernel,
        grid=(M // BM,),
        in_specs=[
            pl.BlockSpec((BM, DI), row),
            pl.BlockSpec((BM, DI), row),
            pl.BlockSpec((BM, C), row),
            pl.BlockSpec((1, DI), fix),
            pl.BlockSpec((1, DI), fix),
            pl.BlockSpec((DI, C), fix),
            pl.BlockSpec((1, C), fix),
            pl.BlockSpec((1, C), fix),
            pl.BlockSpec((C, HID), fix),
            pl.BlockSpec((1, HID), fix),
            pl.BlockSpec((HID, C), fix),
            pl.BlockSpec((1, C), fix),
        ],
        out_specs=pl.BlockSpec((BM, C), row),
        out_shape=jax.ShapeDtypeStruct((M, C), jnp.float32),
        compiler_params=pltpu.CompilerParams(
            dimension_semantics=("parallel",)),
        name="mamba_post",
    )(y, z, sc, ong, onb, wout_T, l2g, l2b, w1_T, b1, w2_T, b2)


@jax.jit
def kernel(x, ln1_g, ln1_b, in_proj_W, conv_W, conv_b, x_proj_W, dt_W, dt_b,
           A_log, Ds, out_norm_g, out_norm_b, out_proj_W, ln2_g, ln2_b,
           fc1_W, fc1_b, fc2_W, fc2_b):
    xin = jnp.transpose(x, (0, 2, 3, 1)).reshape(B, L, C)
    cw9 = jnp.zeros((16, DI), jnp.float32).at[:9].set(
        conv_W.reshape(DI, 9).T)
    xc, z = _pre(xin, ln1_g.reshape(1, C), ln1_b.reshape(1, C),
                 in_proj_W.T, cw9, conv_b.reshape(1, DI))

    xc4 = xc.reshape(B, H, W, DI)
    hwT = jnp.transpose(xc, (1, 0, 2))
    whT = jnp.transpose(xc4, (2, 1, 0, 3)).reshape(L, B, DI)
    xs4 = jnp.stack([hwT, whT, hwT[::-1], whT[::-1]], 0)

    wdt_T = jnp.transpose(x_proj_W[:, :R, :], (0, 2, 1))
    wbc_T = jnp.transpose(x_proj_W[:, R:, :], (0, 2, 1))
    dtw_T = jnp.transpose(dt_W, (0, 2, 1))
    alT = jnp.transpose(A_log, (0, 2, 1))

    ys4 = _scan(xs4, wdt_T, wbc_T, dtw_T, dt_b.reshape(K, 1, DI),
                alT, Ds.reshape(K, 1, DI))

    y_hw = ys4[0] + ys4[2][::-1]
    y_wh = ys4[1] + ys4[3][::-1]
    y_whT = jnp.transpose(y_wh.reshape(W, H, B, DI), (1, 0, 2, 3))
    y = (y_hw + y_whT.reshape(L, B, DI))
    y = jnp.transpose(y, (1, 0, 2)).reshape(M, DI)

    out = _post(y, z.reshape(M, DI), xin.reshape(M, C),
                out_norm_g.reshape(1, DI), out_norm_b.reshape(1, DI),
                out_proj_W.T, ln2_g.reshape(1, C), ln2_b.reshape(1, C),
                fc1_W.T, fc1_b.reshape(1, HID), fc2_W.T, fc2_b.reshape(1, C))
    return out.reshape(B, H, W, C)


# speedup vs baseline: 1.4210x; 1.4210x over previous
import jax
import jax.numpy as jnp
from jax.experimental import pallas as pl
from jax.experimental.pallas import tpu as pltpu

B, C, H, W = 8, 384, 32, 32
L = H * W
N = 4
DI = 2 * C
R = -(-C // 16)
K = 4
HID = 4 * C
LC = 128
NC = L // LC


def _ln(x, g, b, eps=1e-5):
    mu = jnp.mean(x, -1, keepdims=True)
    var = jnp.mean((x - mu) ** 2, -1, keepdims=True)
    return (x - mu) * jax.lax.rsqrt(var + eps) * g + b


def _pre_kernel(xin_ref, g_ref, b_ref, wip_ref, cw_ref, cb_ref,
                xs_ref, z_ref, pad_ref):
    xb = xin_ref[0]
    xn = _ln(xb, g_ref[0], b_ref[0])
    xz = jnp.dot(xn, wip_ref[...], preferred_element_type=jnp.float32)
    z_ref[0] = xz[:, DI:]
    pad_ref[...] = jnp.zeros_like(pad_ref)
    pad_ref[1:H + 1, 1:W + 1, :] = xz[:, :DI].reshape(H, W, DI)
    y = cb_ref[0] * jnp.ones((H, W, DI), jnp.float32)
    for i in range(3):
        for j in range(3):
            y = y + pad_ref[i:i + H, j:j + W, :] * cw_ref[i * 3 + j]
    xc = y * jax.nn.sigmoid(y)
    xs_ref[0, 0] = xc.reshape(L, DI)
    xs_ref[1, 0] = jnp.transpose(xc, (1, 0, 2)).reshape(L, DI)


def _pre(xin, ln1_g, ln1_b, wip_T, cw9, cb):
    return pl.pallas_call(
        _pre_kernel,
        grid=(B,),
        in_specs=[
            pl.BlockSpec((1, L, C), lambda i: (i, 0, 0)),
            pl.BlockSpec((1, C), lambda i: (0, 0)),
            pl.BlockSpec((1, C), lambda i: (0, 0)),
            pl.BlockSpec((C, 2 * DI), lambda i: (0, 0)),
            pl.BlockSpec((16, DI), lambda i: (0, 0)),
            pl.BlockSpec((1, DI), lambda i: (0, 0)),
        ],
        out_specs=[
            pl.BlockSpec((2, 1, L, DI), lambda i: (0, i, 0, 0)),
            pl.BlockSpec((1, L, DI), lambda i: (i, 0, 0)),
        ],
        out_shape=[
            jax.ShapeDtypeStruct((2, B, L, DI), jnp.float32),
            jax.ShapeDtypeStruct((B, L, DI), jnp.float32),
        ],
        scratch_shapes=[pltpu.VMEM((H + 2, W + 2, DI), jnp.float32)],
        compiler_params=pltpu.CompilerParams(
            dimension_semantics=("parallel",)),
        name="mamba_pre",
    )(xin, ln1_g, ln1_b, wip_T, cw9, cb)


def _scan_kernel(x_ref, wdt_ref, wbc_ref, dtw_ref, dtb_ref, al_ref, ds_ref,
                 y_ref, h_ref, xt_ref, dt_ref, bc_ref, yw_ref):
    c = pl.program_id(1)
    rev = c >= NC
    xt_ref[...] = jnp.transpose(x_ref[0], (1, 0, 2))
    xblk = xt_ref[...]
    x2 = xblk.reshape(LC * B, DI)
    dts = jnp.dot(x2, wdt_ref[0, 0], preferred_element_type=jnp.float32)
    dt = jax.nn.softplus(
        jnp.dot(dts, dtw_ref[0, 0], preferred_element_type=jnp.float32)
        + dtb_ref[0, 0])
    dt_ref[...] = dt.reshape(LC, B, DI)
    bc = jnp.dot(x2, wbc_ref[0, 0], preferred_element_type=jnp.float32)
    bc_ref[...] = bc.reshape(LC, B, 2 * N)
    A = -jnp.exp(al_ref[0, 0])

    @pl.when((c == 0) | (c == NC))
    def _():
        h_ref[...] = jnp.zeros_like(h_ref)

    def body(t, hs):
        te = jnp.where(rev, LC - 1 - t, t)
        xt = xt_ref[te]
        dtt = dt_ref[te]
        bct = bc_ref[te]
        du = dtt * xt
        acc = jnp.zeros((B, DI), jnp.float32)
        new = []
        for n in range(N):
            a = jnp.exp(dtt * A[n:n + 1])
            hn = a * hs[n] + du * bct[:, n:n + 1]
            acc = acc + hn * bct[:, N + n:N + n + 1]
            new.append(hn)
        yw_ref[te] = acc
        return tuple(new)

    hs0 = tuple(h_ref[n] for n in range(N))
    hs = jax.lax.fori_loop(0, LC, body, hs0)
    for n in range(N):
        h_ref[n] = hs[n]
    y_ref[0] = jnp.transpose(yw_ref[...] + xblk * ds_ref[0, 0], (1, 0, 2))


def _scan(xs2, wdt_T, wbc_T, dtw_T, dtb2, alT, ds2):
    cmap = lambda c: jnp.where(c < NC, c, 2 * NC - 1 - c)
    dmap = lambda c: c // NC
    return pl.pallas_call(
        _scan_kernel,
        grid=(2, 2 * NC),
        in_specs=[
            pl.BlockSpec((1, B, LC, DI), lambda k, c: (k, 0, cmap(c), 0)),
            pl.BlockSpec((1, 1, DI, R), lambda k, c: (k, dmap(c), 0, 0)),
            pl.BlockSpec((1, 1, DI, 2 * N), lambda k, c: (k, dmap(c), 0, 0)),
            pl.BlockSpec((1, 1, R, DI), lambda k, c: (k, dmap(c), 0, 0)),
            pl.BlockSpec((1, 1, 1, DI), lambda k, c: (k, dmap(c), 0, 0)),
            pl.BlockSpec((1, 1, N, DI), lambda k, c: (k, dmap(c), 0, 0)),
            pl.BlockSpec((1, 1, 1, DI), lambda k, c: (k, dmap(c), 0, 0)),
        ],
        out_specs=pl.BlockSpec(
            (1, B, LC, DI),
            lambda k, c: (k + 2 * dmap(c), 0, cmap(c), 0)),
        out_shape=jax.ShapeDtypeStruct((K, B, L, DI), jnp.float32),
        scratch_shapes=[
            pltpu.VMEM((N, B, DI), jnp.float32),
            pltpu.VMEM((LC, B, DI), jnp.float32),
            pltpu.VMEM((LC, B, DI), jnp.float32),
            pltpu.VMEM((LC, B, 2 * N), jnp.float32),
            pltpu.VMEM((LC, B, DI), jnp.float32),
        ],
        compiler_params=pltpu.CompilerParams(
            dimension_semantics=("parallel", "arbitrary")),
        name="mamba_scan",
    )(xs2, wdt_T, wbc_T, dtw_T, dtb2, alT, ds2)


def _post_kernel(ys_ref, z_ref, sc_ref, ong_ref, onb_ref, wout_ref,
                 l2g_ref, l2b_ref, w1_ref, b1_ref, w2_ref, b2_ref, o_ref):
    y0 = ys_ref[0, 0] + ys_ref[2, 0]
    y1 = ys_ref[1, 0] + ys_ref[3, 0]
    y1t = jnp.transpose(y1.reshape(W, H, DI), (1, 0, 2)).reshape(L, DI)
    y = _ln(y0 + y1t, ong_ref[0], onb_ref[0])
    z = z_ref[0]
    y = y * (z * jax.nn.sigmoid(z))
    u = jnp.dot(y, wout_ref[...], preferred_element_type=jnp.float32)
    xr = sc_ref[0] + u
    m = _ln(xr, l2g_ref[0], l2b_ref[0])
    m = jax.nn.gelu(
        jnp.dot(m, w1_ref[...], preferred_element_type=jnp.float32)
        + b1_ref[0])
    m = jnp.dot(m, w2_ref[...], preferred_element_type=jnp.float32) + b2_ref[0]
    o_ref[0] = xr + m


def _post(ys, z, sc, ong, onb, wout_T, l2g, l2b, w1_T, b1, w2_T, b2):
    fix = lambda j: (0, 0)
    return pl.pallas_call(
        _post_kernel,
        grid=(B,),
        in_specs=[
            pl.BlockSpec((K, 1, L, DI), lambda j: (0, j, 0, 0)),
            pl.BlockSpec((1, L, DI), lambda j: (j, 0, 0)),
            pl.BlockSpec((1, L, C), lambda j: (j, 0, 0)),
            pl.BlockSpec((1, DI), fix),
            pl.BlockSpec((1, DI), fix),
            pl.BlockSpec((DI, C), fix),
            pl.BlockSpec((1, C), fix),
            pl.BlockSpec((1, C), fix),
            pl.BlockSpec((C, HID), fix),
            pl.BlockSpec((1, HID), fix),
            pl.BlockSpec((HID, C), fix),
            pl.BlockSpec((1, C), fix),
        ],
        out_specs=pl.BlockSpec((1, L, C), lambda j: (j, 0, 0)),
        out_shape=jax.ShapeDtypeStruct((B, L, C), jnp.float32),
        compiler_params=pltpu.CompilerParams(
            dimension_semantics=("parallel",)),
        name="mamba_post",
    )(ys, z, sc, ong, onb, wout_T, l2g, l2b, w1_T, b1, w2_T, b2)


@jax.jit
def kernel(x, ln1_g, ln1_b, in_proj_W, conv_W, conv_b, x_proj_W, dt_W, dt_b,
           A_log, Ds, out_norm_g, out_norm_b, out_proj_W, ln2_g, ln2_b,
           fc1_W, fc1_b, fc2_W, fc2_b):
    xin = jnp.transpose(x, (0, 2, 3, 1)).reshape(B, L, C)
    cw9 = jnp.zeros((16, DI), jnp.float32).at[:9].set(
        conv_W.reshape(DI, 9).T)
    xs2, z = _pre(xin, ln1_g.reshape(1, C), ln1_b.reshape(1, C),
                  in_proj_W.T, cw9, conv_b.reshape(1, DI))

    pair = lambda w: jnp.stack([w[:2], w[2:]], 1)
    wdt_T = pair(jnp.transpose(x_proj_W[:, :R, :], (0, 2, 1)))
    wbc_T = pair(jnp.transpose(x_proj_W[:, R:, :], (0, 2, 1)))
    dtw_T = pair(jnp.transpose(dt_W, (0, 2, 1)))
    alT = pair(jnp.transpose(A_log, (0, 2, 1)))

    ys = _scan(xs2, wdt_T, wbc_T, dtw_T,
               pair(dt_b).reshape(2, 2, 1, DI),
               alT, pair(Ds).reshape(2, 2, 1, DI))

    out = _post(ys, z, xin,
                out_norm_g.reshape(1, DI), out_norm_b.reshape(1, DI),
                out_proj_W.T, ln2_g.reshape(1, C), ln2_b.reshape(1, C),
                fc1_W.T, fc1_b.reshape(1, HID), fc2_W.T, fc2_b.reshape(1, C))
    return out.reshape(B, H, W, C)


# speedup vs baseline: 1.8148x; 1.2771x over previous
import jax
import jax.numpy as jnp
from jax.experimental import pallas as pl
from jax.experimental.pallas import tpu as pltpu

B, C, H, W = 8, 384, 32, 32
L = H * W
N = 4
DI = 2 * C
R = -(-C // 16)
K = 4
HID = 4 * C
LC = 256
NC = L // LC


def _ln(x, g, b, eps=1e-5):
    mu = jnp.mean(x, -1, keepdims=True)
    var = jnp.mean((x - mu) ** 2, -1, keepdims=True)
    return (x - mu) * jax.lax.rsqrt(var + eps) * g + b


def _pre_kernel(xin_ref, g_ref, b_ref, wip_ref, cw_ref, cb_ref,
                xs_ref, z_ref, pad_ref):
    xb = xin_ref[0]
    xn = _ln(xb, g_ref[0], b_ref[0])
    xz = jnp.dot(xn, wip_ref[...], preferred_element_type=jnp.float32)
    z_ref[0] = xz[:, DI:]
    pad_ref[...] = jnp.zeros_like(pad_ref)
    pad_ref[1:H + 1, 1:W + 1, :] = xz[:, :DI].reshape(H, W, DI)
    y = cb_ref[0] * jnp.ones((H, W, DI), jnp.float32)
    for i in range(3):
        for j in range(3):
            y = y + pad_ref[i:i + H, j:j + W, :] * cw_ref[i * 3 + j]
    xc = y * jax.nn.sigmoid(y)
    xs_ref[0, 0] = xc.reshape(L, DI)
    xs_ref[1, 0] = jnp.transpose(xc, (1, 0, 2)).reshape(L, DI)


def _pre(xin, ln1_g, ln1_b, wip_T, cw9, cb):
    return pl.pallas_call(
        _pre_kernel,
        grid=(B,),
        in_specs=[
            pl.BlockSpec((1, L, C), lambda i: (i, 0, 0)),
            pl.BlockSpec((1, C), lambda i: (0, 0)),
            pl.BlockSpec((1, C), lambda i: (0, 0)),
            pl.BlockSpec((C, 2 * DI), lambda i: (0, 0)),
            pl.BlockSpec((16, DI), lambda i: (0, 0)),
            pl.BlockSpec((1, DI), lambda i: (0, 0)),
        ],
        out_specs=[
            pl.BlockSpec((2, 1, L, DI), lambda i: (0, i, 0, 0)),
            pl.BlockSpec((1, L, DI), lambda i: (i, 0, 0)),
        ],
        out_shape=[
            jax.ShapeDtypeStruct((2, B, L, DI), jnp.float32),
            jax.ShapeDtypeStruct((B, L, DI), jnp.float32),
        ],
        scratch_shapes=[pltpu.VMEM((H + 2, W + 2, DI), jnp.float32)],
        compiler_params=pltpu.CompilerParams(
            dimension_semantics=("parallel",)),
        name="mamba_pre",
    )(xin, ln1_g, ln1_b, wip_T, cw9, cb)


def _scan_kernel(x_ref, wdt_ref, wbc_ref, dtw_ref, dtb_ref, al_ref, ds_ref,
                 y_ref, h_ref, xt_ref, dt_ref, bc_ref, yw_ref):
    c = pl.program_id(1)
    rev = c >= NC
    xt_ref[...] = jnp.transpose(x_ref[0], (1, 0, 2))
    xblk = xt_ref[...]
    x2 = xblk.reshape(LC * B, DI)
    dts = jnp.dot(x2, wdt_ref[0, 0], preferred_element_type=jnp.float32)
    dt = jax.nn.softplus(
        jnp.dot(dts, dtw_ref[0, 0], preferred_element_type=jnp.float32)
        + dtb_ref[0, 0])
    dt_ref[...] = dt.reshape(LC, B, DI)
    bc = jnp.dot(x2, wbc_ref[0, 0], preferred_element_type=jnp.float32)
    bc_ref[...] = bc.reshape(LC, B, 2 * N)
    A = -jnp.exp(al_ref[0, 0])

    @pl.when((c == 0) | (c == NC))
    def _():
        h_ref[...] = jnp.zeros_like(h_ref)

    def body(t, hs):
        te = jnp.where(rev, LC - 1 - t, t)
        xt = xt_ref[te]
        dtt = dt_ref[te]
        bct = bc_ref[te]
        du = dtt * xt
        acc = jnp.zeros((B, DI), jnp.float32)
        new = []
        for n in range(N):
            a = jnp.exp(dtt * A[n:n + 1])
            hn = a * hs[n] + du * bct[:, n:n + 1]
            acc = acc + hn * bct[:, N + n:N + n + 1]
            new.append(hn)
        yw_ref[te] = acc
        return tuple(new)

    hs0 = tuple(h_ref[n] for n in range(N))
    hs = jax.lax.fori_loop(0, LC, body, hs0, unroll=2)
    for n in range(N):
        h_ref[n] = hs[n]
    y_ref[0] = jnp.transpose(yw_ref[...] + xblk * ds_ref[0, 0], (1, 0, 2))


def _scan(xs2, wdt_T, wbc_T, dtw_T, dtb2, alT, ds2):
    cmap = lambda c: jnp.where(c < NC, c, 2 * NC - 1 - c)
    dmap = lambda c: c // NC
    return pl.pallas_call(
        _scan_kernel,
        grid=(2, 2 * NC),
        in_specs=[
            pl.BlockSpec((1, B, LC, DI), lambda k, c: (k, 0, cmap(c), 0)),
            pl.BlockSpec((1, 1, DI, R), lambda k, c: (k, dmap(c), 0, 0)),
            pl.BlockSpec((1, 1, DI, 2 * N), lambda k, c: (k, dmap(c), 0, 0)),
            pl.BlockSpec((1, 1, R, DI), lambda k, c: (k, dmap(c), 0, 0)),
            pl.BlockSpec((1, 1, 1, DI), lambda k, c: (k, dmap(c), 0, 0)),
            pl.BlockSpec((1, 1, N, DI), lambda k, c: (k, dmap(c), 0, 0)),
            pl.BlockSpec((1, 1, 1, DI), lambda k, c: (k, dmap(c), 0, 0)),
        ],
        out_specs=pl.BlockSpec(
            (1, B, LC, DI),
            lambda k, c: (k + 2 * dmap(c), 0, cmap(c), 0)),
        out_shape=jax.ShapeDtypeStruct((K, B, L, DI), jnp.float32),
        scratch_shapes=[
            pltpu.VMEM((N, B, DI), jnp.float32),
            pltpu.VMEM((LC, B, DI), jnp.float32),
            pltpu.VMEM((LC, B, DI), jnp.float32),
            pltpu.VMEM((LC, B, 2 * N), jnp.float32),
            pltpu.VMEM((LC, B, DI), jnp.float32),
        ],
        compiler_params=pltpu.CompilerParams(
            dimension_semantics=("parallel", "arbitrary")),
        name="mamba_scan",
    )(xs2, wdt_T, wbc_T, dtw_T, dtb2, alT, ds2)


def _post_kernel(ys_ref, z_ref, sc_ref, ong_ref, onb_ref, wout_ref,
                 l2g_ref, l2b_ref, w1_ref, b1_ref, w2_ref, b2_ref, o_ref):
    y0 = ys_ref[0, 0] + ys_ref[2, 0]
    y1 = ys_ref[1, 0] + ys_ref[3, 0]
    y1t = jnp.transpose(y1.reshape(W, H, DI), (1, 0, 2)).reshape(L, DI)
    y = _ln(y0 + y1t, ong_ref[0], onb_ref[0])
    z = z_ref[0]
    y = y * (z * jax.nn.sigmoid(z))
    u = jnp.dot(y, wout_ref[...], preferred_element_type=jnp.float32)
    xr = sc_ref[0] + u
    m = _ln(xr, l2g_ref[0], l2b_ref[0])
    m = jax.nn.gelu(
        jnp.dot(m, w1_ref[...], preferred_element_type=jnp.float32)
        + b1_ref[0])
    m = jnp.dot(m, w2_ref[...], preferred_element_type=jnp.float32) + b2_ref[0]
    o_ref[0] = xr + m


def _post(ys, z, sc, ong, onb, wout_T, l2g, l2b, w1_T, b1, w2_T, b2):
    fix = lambda j: (0, 0)
    return pl.pallas_call(
        _post_kernel,
        grid=(B,),
        in_specs=[
            pl.BlockSpec((K, 1, L, DI), lambda j: (0, j, 0, 0)),
            pl.BlockSpec((1, L, DI), lambda j: (j, 0, 0)),
            pl.BlockSpec((1, L, C), lambda j: (j, 0, 0)),
            pl.BlockSpec((1, DI), fix),
            pl.BlockSpec((1, DI), fix),
            pl.BlockSpec((DI, C), fix),
            pl.BlockSpec((1, C), fix),
            pl.BlockSpec((1, C), fix),
            pl.BlockSpec((C, HID), fix),
            pl.BlockSpec((1, HID), fix),
            pl.BlockSpec((HID, C), fix),
            pl.BlockSpec((1, C), fix),
        ],
        out_specs=pl.BlockSpec((1, L, C), lambda j: (j, 0, 0)),
        out_shape=jax.ShapeDtypeStruct((B, L, C), jnp.float32),
        compiler_params=pltpu.CompilerParams(
            dimension_semantics=("parallel",)),
        name="mamba_post",
    )(ys, z, sc, ong, onb, wout_T, l2g, l2b, w1_T, b1, w2_T, b2)


@jax.jit
def kernel(x, ln1_g, ln1_b, in_proj_W, conv_W, conv_b, x_proj_W, dt_W, dt_b,
           A_log, Ds, out_norm_g, out_norm_b, out_proj_W, ln2_g, ln2_b,
           fc1_W, fc1_b, fc2_W, fc2_b):
    xin = jnp.transpose(x, (0, 2, 3, 1)).reshape(B, L, C)
    cw9 = jnp.zeros((16, DI), jnp.float32).at[:9].set(
        conv_W.reshape(DI, 9).T)
    xs2, z = _pre(xin, ln1_g.reshape(1, C), ln1_b.reshape(1, C),
                  in_proj_W.T, cw9, conv_b.reshape(1, DI))

    pair = lambda w: jnp.stack([w[:2], w[2:]], 1)
    wdt_T = pair(jnp.transpose(x_proj_W[:, :R, :], (0, 2, 1)))
    wbc_T = pair(jnp.transpose(x_proj_W[:, R:, :], (0, 2, 1)))
    dtw_T = pair(jnp.transpose(dt_W, (0, 2, 1)))
    alT = pair(jnp.transpose(A_log, (0, 2, 1)))

    ys = _scan(xs2, wdt_T, wbc_T, dtw_T,
               pair(dt_b).reshape(2, 2, 1, DI),
               alT, pair(Ds).reshape(2, 2, 1, DI))

    out = _post(ys, z, xin,
                out_norm_g.reshape(1, DI), out_norm_b.reshape(1, DI),
                out_proj_W.T, ln2_g.reshape(1, C), ln2_b.reshape(1, C),
                fc1_W.T, fc1_b.reshape(1, HID), fc2_W.T, fc2_b.reshape(1, C))
    return out.reshape(B, H, W, C)


# speedup vs baseline: 2.1006x; 1.1575x over previous
import jax
import jax.numpy as jnp
from jax.experimental import pallas as pl
from jax.experimental.pallas import tpu as pltpu

B, C, H, W = 8, 384, 32, 32
L = H * W
N = 4
DI = 2 * C
R = -(-C // 16)
K = 4
HID = 4 * C
LC = 256
NC = L // LC


def _ln(x, g, b, eps=1e-5):
    mu = jnp.mean(x, -1, keepdims=True)
    var = jnp.mean((x - mu) ** 2, -1, keepdims=True)
    return (x - mu) * jax.lax.rsqrt(var + eps) * g + b


def _pre_kernel(xin_ref, g_ref, b_ref, wip_ref, cw_ref, cb_ref,
                xs_ref, z_ref, pad_ref):
    xb = xin_ref[0]
    xn = _ln(xb, g_ref[0], b_ref[0])
    xz = jnp.dot(xn, wip_ref[...], preferred_element_type=jnp.float32)
    z_ref[0] = xz[:, DI:]
    pad_ref[...] = jnp.zeros_like(pad_ref)
    pad_ref[1:H + 1, 1:W + 1, :] = xz[:, :DI].reshape(H, W, DI)
    y = cb_ref[0] * jnp.ones((H, W, DI), jnp.float32)
    for i in range(3):
        for j in range(3):
            y = y + pad_ref[i:i + H, j:j + W, :] * cw_ref[i * 3 + j]
    xc = y * jax.nn.sigmoid(y)
    xs_ref[0, 0] = xc.reshape(L, DI)
    xs_ref[1, 0] = jnp.transpose(xc, (1, 0, 2)).reshape(L, DI)


def _pre(xin, ln1_g, ln1_b, wip_T, cw9, cb):
    return pl.pallas_call(
        _pre_kernel,
        grid=(B,),
        in_specs=[
            pl.BlockSpec((1, L, C), lambda i: (i, 0, 0)),
            pl.BlockSpec((1, C), lambda i: (0, 0)),
            pl.BlockSpec((1, C), lambda i: (0, 0)),
            pl.BlockSpec((C, 2 * DI), lambda i: (0, 0)),
            pl.BlockSpec((16, DI), lambda i: (0, 0)),
            pl.BlockSpec((1, DI), lambda i: (0, 0)),
        ],
        out_specs=[
            pl.BlockSpec((2, 1, L, DI), lambda i: (0, i, 0, 0)),
            pl.BlockSpec((1, L, DI), lambda i: (i, 0, 0)),
        ],
        out_shape=[
            jax.ShapeDtypeStruct((2, B, L, DI), jnp.float32),
            jax.ShapeDtypeStruct((B, L, DI), jnp.float32),
        ],
        scratch_shapes=[pltpu.VMEM((H + 2, W + 2, DI), jnp.float32)],
        compiler_params=pltpu.CompilerParams(
            dimension_semantics=("parallel",)),
        name="mamba_pre",
    )(xin, ln1_g, ln1_b, wip_T, cw9, cb)


def _scan_kernel(x_ref, wdt_ref, wbc_ref, dtw_ref, dtb_ref, al_ref, ds_ref,
                 y_ref, h_ref, xt_ref, dt_ref, bc_ref, yw_ref):
    c = pl.program_id(1)
    rev = c >= NC
    xt_ref[...] = jnp.transpose(x_ref[0], (1, 0, 2))
    xblk = xt_ref[...]
    x2 = xblk.reshape(LC * B, DI)
    dts = jnp.dot(x2, wdt_ref[0, 0], preferred_element_type=jnp.float32)
    dt = jax.nn.softplus(
        jnp.dot(dts, dtw_ref[0, 0], preferred_element_type=jnp.float32)
        + dtb_ref[0, 0])
    dt_ref[...] = dt.reshape(LC, B, DI)
    bc = jnp.dot(x2, wbc_ref[0, 0], preferred_element_type=jnp.float32)
    bc_ref[...] = bc.reshape(LC, B, 2 * N)
    A = -jnp.exp(al_ref[0, 0])

    @pl.when((c == 0) | (c == NC))
    def _():
        h_ref[...] = jnp.zeros_like(h_ref)

    def body(t, hs):
        te = jnp.where(rev, LC - 1 - t, t)
        xt = xt_ref[te]
        dtt = dt_ref[te]
        bct = bc_ref[te]
        du = dtt * xt
        acc = jnp.zeros((B, DI), jnp.float32)
        new = []
        for n in range(N):
            a = jnp.exp(dtt * A[n:n + 1])
            hn = a * hs[n] + du * bct[:, n:n + 1]
            acc = acc + hn * bct[:, N + n:N + n + 1]
            new.append(hn)
        yw_ref[te] = acc
        return tuple(new)

    hs0 = tuple(h_ref[n] for n in range(N))
    hs = jax.lax.fori_loop(0, LC, body, hs0, unroll=4)
    for n in range(N):
        h_ref[n] = hs[n]
    y_ref[0] = jnp.transpose(yw_ref[...] + xblk * ds_ref[0, 0], (1, 0, 2))


def _scan(xs2, wdt_T, wbc_T, dtw_T, dtb2, alT, ds2):
    cmap = lambda c: jnp.where(c < NC, c, 2 * NC - 1 - c)
    dmap = lambda c: c // NC
    return pl.pallas_call(
        _scan_kernel,
        grid=(2, 2 * NC),
        in_specs=[
            pl.BlockSpec((1, B, LC, DI), lambda k, c: (k, 0, cmap(c), 0)),
            pl.BlockSpec((1, 1, DI, R), lambda k, c: (k, dmap(c), 0, 0)),
            pl.BlockSpec((1, 1, DI, 2 * N), lambda k, c: (k, dmap(c), 0, 0)),
            pl.BlockSpec((1, 1, R, DI), lambda k, c: (k, dmap(c), 0, 0)),
            pl.BlockSpec((1, 1, 1, DI), lambda k, c: (k, dmap(c), 0, 0)),
            pl.BlockSpec((1, 1, N, DI), lambda k, c: (k, dmap(c), 0, 0)),
            pl.BlockSpec((1, 1, 1, DI), lambda k, c: (k, dmap(c), 0, 0)),
        ],
        out_specs=pl.BlockSpec(
            (1, B, LC, DI),
            lambda k, c: (k + 2 * dmap(c), 0, cmap(c), 0)),
        out_shape=jax.ShapeDtypeStruct((K, B, L, DI), jnp.float32),
        scratch_shapes=[
            pltpu.VMEM((N, B, DI), jnp.float32),
            pltpu.VMEM((LC, B, DI), jnp.float32),
            pltpu.VMEM((LC, B, DI), jnp.float32),
            pltpu.VMEM((LC, B, 2 * N), jnp.float32),
            pltpu.VMEM((LC, B, DI), jnp.float32),
        ],
        compiler_params=pltpu.CompilerParams(
            dimension_semantics=("parallel", "arbitrary")),
        name="mamba_scan",
    )(xs2, wdt_T, wbc_T, dtw_T, dtb2, alT, ds2)


def _post_kernel(ys_ref, z_ref, sc_ref, ong_ref, onb_ref, wout_ref,
                 l2g_ref, l2b_ref, w1_ref, b1_ref, w2_ref, b2_ref, o_ref):
    y0 = ys_ref[0, 0] + ys_ref[2, 0]
    y1 = ys_ref[1, 0] + ys_ref[3, 0]
    y1t = jnp.transpose(y1.reshape(W, H, DI), (1, 0, 2)).reshape(L, DI)
    y = _ln(y0 + y1t, ong_ref[0], onb_ref[0])
    z = z_ref[0]
    y = y * (z * jax.nn.sigmoid(z))
    u = jnp.dot(y, wout_ref[...], preferred_element_type=jnp.float32)
    xr = sc_ref[0] + u
    m = _ln(xr, l2g_ref[0], l2b_ref[0])
    m = jax.nn.gelu(
        jnp.dot(m, w1_ref[...], preferred_element_type=jnp.float32)
        + b1_ref[0])
    m = jnp.dot(m, w2_ref[...], preferred_element_type=jnp.float32) + b2_ref[0]
    o_ref[0] = xr + m


def _post(ys, z, sc, ong, onb, wout_T, l2g, l2b, w1_T, b1, w2_T, b2):
    fix = lambda j: (0, 0)
    return pl.pallas_call(
        _post_kernel,
        grid=(B,),
        in_specs=[
            pl.BlockSpec((K, 1, L, DI), lambda j: (0, j, 0, 0)),
            pl.BlockSpec((1, L, DI), lambda j: (j, 0, 0)),
            pl.BlockSpec((1, L, C), lambda j: (j, 0, 0)),
            pl.BlockSpec((1, DI), fix),
            pl.BlockSpec((1, DI), fix),
            pl.BlockSpec((DI, C), fix),
            pl.BlockSpec((1, C), fix),
            pl.BlockSpec((1, C), fix),
            pl.BlockSpec((C, HID), fix),
            pl.BlockSpec((1, HID), fix),
            pl.BlockSpec((HID, C), fix),
            pl.BlockSpec((1, C), fix),
        ],
        out_specs=pl.BlockSpec((1, L, C), lambda j: (j, 0, 0)),
        out_shape=jax.ShapeDtypeStruct((B, L, C), jnp.float32),
        compiler_params=pltpu.CompilerParams(
            dimension_semantics=("parallel",)),
        name="mamba_post",
    )(ys, z, sc, ong, onb, wout_T, l2g, l2b, w1_T, b1, w2_T, b2)


@jax.jit
def kernel(x, ln1_g, ln1_b, in_proj_W, conv_W, conv_b, x_proj_W, dt_W, dt_b,
           A_log, Ds, out_norm_g, out_norm_b, out_proj_W, ln2_g, ln2_b,
           fc1_W, fc1_b, fc2_W, fc2_b):
    xin = jnp.transpose(x, (0, 2, 3, 1)).reshape(B, L, C)
    cw9 = jnp.zeros((16, DI), jnp.float32).at[:9].set(
        conv_W.reshape(DI, 9).T)
    xs2, z = _pre(xin, ln1_g.reshape(1, C), ln1_b.reshape(1, C),
                  in_proj_W.T, cw9, conv_b.reshape(1, DI))

    pair = lambda w: jnp.stack([w[:2], w[2:]], 1)
    wdt_T = pair(jnp.transpose(x_proj_W[:, :R, :], (0, 2, 1)))
    wbc_T = pair(jnp.transpose(x_proj_W[:, R:, :], (0, 2, 1)))
    dtw_T = pair(jnp.transpose(dt_W, (0, 2, 1)))
    alT = pair(jnp.transpose(A_log, (0, 2, 1)))

    ys = _scan(xs2, wdt_T, wbc_T, dtw_T,
               pair(dt_b).reshape(2, 2, 1, DI),
               alT, pair(Ds).reshape(2, 2, 1, DI))

    out = _post(ys, z, xin,
                out_norm_g.reshape(1, DI), out_norm_b.reshape(1, DI),
                out_proj_W.T, ln2_g.reshape(1, C), ln2_b.reshape(1, C),
                fc1_W.T, fc1_b.reshape(1, HID), fc2_W.T, fc2_b.reshape(1, C))
    return out.reshape(B, H, W, C)


# speedup vs baseline: 2.1844x; 1.0399x over previous
import jax
import jax.numpy as jnp
from jax.experimental import pallas as pl
from jax.experimental.pallas import tpu as pltpu

B, C, H, W = 8, 384, 32, 32
L = H * W
N = 4
DI = 2 * C
R = -(-C // 16)
K = 4
HID = 4 * C
LC = 256
NC = L // LC


def _ln(x, g, b, eps=1e-5):
    mu = jnp.mean(x, -1, keepdims=True)
    var = jnp.mean((x - mu) ** 2, -1, keepdims=True)
    return (x - mu) * jax.lax.rsqrt(var + eps) * g + b


def _pre_kernel(xin_ref, g_ref, b_ref, wip_ref, cw_ref, cb_ref,
                xs_ref, z_ref, pad_ref):
    xb = xin_ref[0]
    xn = _ln(xb, g_ref[0], b_ref[0])
    xz = jnp.dot(xn, wip_ref[...], preferred_element_type=jnp.float32)
    z_ref[0] = xz[:, DI:]
    pad_ref[...] = jnp.zeros_like(pad_ref)
    pad_ref[1:H + 1, 1:W + 1, :] = xz[:, :DI].reshape(H, W, DI)
    y = cb_ref[0] * jnp.ones((H, W, DI), jnp.float32)
    for i in range(3):
        for j in range(3):
            y = y + pad_ref[i:i + H, j:j + W, :] * cw_ref[i * 3 + j]
    xc = y * jax.nn.sigmoid(y)
    xs_ref[0, 0] = xc.reshape(L, DI)
    xs_ref[1, 0] = jnp.transpose(xc, (1, 0, 2)).reshape(L, DI)


def _pre(xin, ln1_g, ln1_b, wip_T, cw9, cb):
    return pl.pallas_call(
        _pre_kernel,
        grid=(B,),
        in_specs=[
            pl.BlockSpec((1, L, C), lambda i: (i, 0, 0)),
            pl.BlockSpec((1, C), lambda i: (0, 0)),
            pl.BlockSpec((1, C), lambda i: (0, 0)),
            pl.BlockSpec((C, 2 * DI), lambda i: (0, 0)),
            pl.BlockSpec((16, DI), lambda i: (0, 0)),
            pl.BlockSpec((1, DI), lambda i: (0, 0)),
        ],
        out_specs=[
            pl.BlockSpec((2, 1, L, DI), lambda i: (0, i, 0, 0)),
            pl.BlockSpec((1, L, DI), lambda i: (i, 0, 0)),
        ],
        out_shape=[
            jax.ShapeDtypeStruct((2, B, L, DI), jnp.float32),
            jax.ShapeDtypeStruct((B, L, DI), jnp.float32),
        ],
        scratch_shapes=[pltpu.VMEM((H + 2, W + 2, DI), jnp.float32)],
        compiler_params=pltpu.CompilerParams(
            dimension_semantics=("parallel",)),
        name="mamba_pre",
    )(xin, ln1_g, ln1_b, wip_T, cw9, cb)


def _scan_kernel(x_ref, wdt_ref, wbc_ref, dtw_ref, dtb_ref, al_ref, ds_ref,
                 y_ref, h_ref, xt_ref, dt_ref, bc_ref, yw_ref):
    c = pl.program_id(1)
    rev = c >= NC
    xt_ref[...] = jnp.transpose(x_ref[0], (1, 0, 2))
    xblk = xt_ref[...]
    x2 = xblk.reshape(LC * B, DI)
    dts = jnp.dot(x2, wdt_ref[0, 0], preferred_element_type=jnp.float32)
    dt = jax.nn.softplus(
        jnp.dot(dts, dtw_ref[0, 0], preferred_element_type=jnp.float32)
        + dtb_ref[0, 0])
    dt_ref[...] = dt.reshape(LC, B, DI)
    bc = jnp.dot(x2, wbc_ref[0, 0], preferred_element_type=jnp.float32)
    bc_ref[...] = bc.reshape(LC, B, 2 * N)
    A = -jnp.exp(al_ref[0, 0])

    @pl.when((c == 0) | (c == NC))
    def _():
        h_ref[...] = jnp.zeros_like(h_ref)

    def body(t, hs):
        te = jnp.where(rev, LC - 1 - t, t)
        xt = xt_ref[te]
        dtt = dt_ref[te]
        bct = bc_ref[te]
        du = dtt * xt
        acc = jnp.zeros((B, DI), jnp.float32)
        new = []
        for n in range(N):
            a = jnp.exp(dtt * A[n:n + 1])
            hn = a * hs[n] + du * bct[:, n:n + 1]
            acc = acc + hn * bct[:, N + n:N + n + 1]
            new.append(hn)
        yw_ref[te] = acc
        return tuple(new)

    hs0 = tuple(h_ref[n] for n in range(N))
    hs = jax.lax.fori_loop(0, LC, body, hs0, unroll=8)
    for n in range(N):
        h_ref[n] = hs[n]
    y_ref[0] = jnp.transpose(yw_ref[...] + xblk * ds_ref[0, 0], (1, 0, 2))


def _scan(xs2, wdt_T, wbc_T, dtw_T, dtb2, alT, ds2):
    cmap = lambda c: jnp.where(c < NC, c, 2 * NC - 1 - c)
    dmap = lambda c: c // NC
    return pl.pallas_call(
        _scan_kernel,
        grid=(2, 2 * NC),
        in_specs=[
            pl.BlockSpec((1, B, LC, DI), lambda k, c: (k, 0, cmap(c), 0)),
            pl.BlockSpec((1, 1, DI, R), lambda k, c: (k, dmap(c), 0, 0)),
            pl.BlockSpec((1, 1, DI, 2 * N), lambda k, c: (k, dmap(c), 0, 0)),
            pl.BlockSpec((1, 1, R, DI), lambda k, c: (k, dmap(c), 0, 0)),
            pl.BlockSpec((1, 1, 1, DI), lambda k, c: (k, dmap(c), 0, 0)),
            pl.BlockSpec((1, 1, N, DI), lambda k, c: (k, dmap(c), 0, 0)),
            pl.BlockSpec((1, 1, 1, DI), lambda k, c: (k, dmap(c), 0, 0)),
        ],
        out_specs=pl.BlockSpec(
            (1, B, LC, DI),
            lambda k, c: (k + 2 * dmap(c), 0, cmap(c), 0)),
        out_shape=jax.ShapeDtypeStruct((K, B, L, DI), jnp.float32),
        scratch_shapes=[
            pltpu.VMEM((N, B, DI), jnp.float32),
            pltpu.VMEM((LC, B, DI), jnp.float32),
            pltpu.VMEM((LC, B, DI), jnp.float32),
            pltpu.VMEM((LC, B, 2 * N), jnp.float32),
            pltpu.VMEM((LC, B, DI), jnp.float32),
        ],
        compiler_params=pltpu.CompilerParams(
            dimension_semantics=("parallel", "arbitrary")),
        name="mamba_scan",
    )(xs2, wdt_T, wbc_T, dtw_T, dtb2, alT, ds2)


def _post_kernel(ys_ref, z_ref, sc_ref, ong_ref, onb_ref, wout_ref,
                 l2g_ref, l2b_ref, w1_ref, b1_ref, w2_ref, b2_ref, o_ref):
    y0 = ys_ref[0, 0] + ys_ref[2, 0]
    y1 = ys_ref[1, 0] + ys_ref[3, 0]
    y1t = jnp.transpose(y1.reshape(W, H, DI), (1, 0, 2)).reshape(L, DI)
    y = _ln(y0 + y1t, ong_ref[0], onb_ref[0])
    z = z_ref[0]
    y = y * (z * jax.nn.sigmoid(z))
    u = jnp.dot(y, wout_ref[...], preferred_element_type=jnp.float32)
    xr = sc_ref[0] + u
    m = _ln(xr, l2g_ref[0], l2b_ref[0])
    m = jax.nn.gelu(
        jnp.dot(m, w1_ref[...], preferred_element_type=jnp.float32)
        + b1_ref[0])
    m = jnp.dot(m, w2_ref[...], preferred_element_type=jnp.float32) + b2_ref[0]
    o_ref[0] = xr + m


def _post(ys, z, sc, ong, onb, wout_T, l2g, l2b, w1_T, b1, w2_T, b2):
    fix = lambda j: (0, 0)
    return pl.pallas_call(
        _post_kernel,
        grid=(B,),
        in_specs=[
            pl.BlockSpec((K, 1, L, DI), lambda j: (0, j, 0, 0)),
            pl.BlockSpec((1, L, DI), lambda j: (j, 0, 0)),
            pl.BlockSpec((1, L, C), lambda j: (j, 0, 0)),
            pl.BlockSpec((1, DI), fix),
            pl.BlockSpec((1, DI), fix),
            pl.BlockSpec((DI, C), fix),
            pl.BlockSpec((1, C), fix),
            pl.BlockSpec((1, C), fix),
            pl.BlockSpec((C, HID), fix),
            pl.BlockSpec((1, HID), fix),
            pl.BlockSpec((HID, C), fix),
            pl.BlockSpec((1, C), fix),
        ],
        out_specs=pl.BlockSpec((1, L, C), lambda j: (j, 0, 0)),
        out_shape=jax.ShapeDtypeStruct((B, L, C), jnp.float32),
        compiler_params=pltpu.CompilerParams(
            dimension_semantics=("parallel",)),
        name="mamba_post",
    )(ys, z, sc, ong, onb, wout_T, l2g, l2b, w1_T, b1, w2_T, b2)


@jax.jit
def kernel(x, ln1_g, ln1_b, in_proj_W, conv_W, conv_b, x_proj_W, dt_W, dt_b,
           A_log, Ds, out_norm_g, out_norm_b, out_proj_W, ln2_g, ln2_b,
           fc1_W, fc1_b, fc2_W, fc2_b):
    xin = jnp.transpose(x, (0, 2, 3, 1)).reshape(B, L, C)
    cw9 = jnp.zeros((16, DI), jnp.float32).at[:9].set(
        conv_W.reshape(DI, 9).T)
    xs2, z = _pre(xin, ln1_g.reshape(1, C), ln1_b.reshape(1, C),
                  in_proj_W.T, cw9, conv_b.reshape(1, DI))

    pair = lambda w: jnp.stack([w[:2], w[2:]], 1)
    wdt_T = pair(jnp.transpose(x_proj_W[:, :R, :], (0, 2, 1)))
    wbc_T = pair(jnp.transpose(x_proj_W[:, R:, :], (0, 2, 1)))
    dtw_T = pair(jnp.transpose(dt_W, (0, 2, 1)))
    alT = pair(jnp.transpose(A_log, (0, 2, 1)))

    ys = _scan(xs2, wdt_T, wbc_T, dtw_T,
               pair(dt_b).reshape(2, 2, 1, DI),
               alT, pair(Ds).reshape(2, 2, 1, DI))

    out = _post(ys, z, xin,
                out_norm_g.reshape(1, DI), out_norm_b.reshape(1, DI),
                out_proj_W.T, ln2_g.reshape(1, C), ln2_b.reshape(1, C),
                fc1_W.T, fc1_b.reshape(1, HID), fc2_W.T, fc2_b.reshape(1, C))
    return out.reshape(B, H, W, C)
